# Initial kernel scaffold; baseline (speedup 1.0000x reference)
import jax
import jax.numpy as jnp
from jax.experimental import pallas as pl


def kernel(input_ids, table):
    raise NotImplementedError("write your pallas kernel here")


# speedup vs baseline: 3.6183x; 3.6183x over previous
import functools

import jax
import jax.numpy as jnp
from jax import lax
from jax.experimental import pallas as pl
from jax.experimental.pallas import tpu as pltpu
from jax.experimental.pallas import tpu_sc as plsc

_MAX_SEQ = 8192
_D = 1024
_BSZ = 4
_NC = 2
_NS = 16
_NW = _NC * _NS
_ROWS_PER_W = _MAX_SEQ // _NW
_CHUNK = 64
_NCHUNK = _ROWS_PER_W // _CHUNK


def _make_bcast():
    mesh = plsc.VectorSubcoreMesh(core_axis_name="c", subcore_axis_name="s")

    @functools.partial(
        pl.kernel,
        mesh=mesh,
        out_type=jax.ShapeDtypeStruct((_BSZ * _MAX_SEQ, _D), jnp.float32),
        scratch_types=[
            pltpu.VMEM((_CHUNK, _D), jnp.float32),
            pltpu.SemaphoreType.DMA,
            pltpu.SemaphoreType.DMA,
        ],
    )
    def bcast(table_hbm, out_hbm, buf, load_sem, store_sem):
        wid = lax.axis_index("s") * _NC + lax.axis_index("c")
        base = wid * _ROWS_PER_W
        for i in range(_NCHUNK):
            off = base + i * _CHUNK
            pltpu.async_copy(table_hbm.at[pl.ds(off, _CHUNK)], buf, load_sem).wait()
            stores = [
                pltpu.async_copy(
                    buf, out_hbm.at[pl.ds(b * _MAX_SEQ + off, _CHUNK)], store_sem
                )
                for b in range(_BSZ)
            ]
            for s in stores:
                s.wait()

    return bcast


_bcast = _make_bcast()


def kernel(input_ids, table):
    del input_ids
    return _bcast(table).reshape(_BSZ, _MAX_SEQ, _D)
